# Initial kernel scaffold; baseline (speedup 1.0000x reference)
import jax
import jax.numpy as jnp
from jax.experimental import pallas as pl


def kernel(user_emb, item_emb, edge_values, edge_index):
    raise NotImplementedError("write your pallas kernel here")


# speedup vs baseline: 11.7122x; 11.7122x over previous
import functools

import jax
import jax.numpy as jnp
from jax import lax
from jax.experimental import pallas as pl
from jax.experimental.pallas import tpu as pltpu
from jax.experimental.pallas import tpu_sc as plsc

_NUM_USERS = 30000
_NUM_ITEMS = 70000
_N = _NUM_USERS + _NUM_ITEMS
_H = 16
_E = 1600000
_NS = 16
_SUB = 128
_CHUNK = 1024
_K = _CHUNK // _SUB
_NCHUNK = -(-_E // (_NS * _CHUNK))
_EPT = _NCHUNK * _CHUNK
_EPAD = _EPT * _NS
_IROWS_PT = _EPT // _SUB
_NP = 102400
_RPT = _NP // _NS
_QROWS = 256
_NQ = _RPT // _QROWS


def _spmm3_body(x0, cols, rows, vals, sum_o, xbuf,
                g_v, idx_v, row_v, val_v, sa_v, sb_v, acc, sem):
    c = lax.axis_index("c")
    s = lax.axis_index("s")
    r0 = s * _RPT

    for l in range(3):
        def zrow(i, _):
            sa_v[i, :] = jnp.zeros((_H,), jnp.float32)
            return 0
        lax.fori_loop(0, _QROWS, zrow, 0)

        def zq(q, _):
            pltpu.sync_copy(sa_v, acc.at[pl.ds(r0 + q * _QROWS, _QROWS)])
            return 0
        lax.fori_loop(0, _NQ, zq, 0)
        plsc.subcore_barrier()

        src = x0 if l == 0 else xbuf

        def chunk(i, _):
            ib = s * _IROWS_PT + i * _K
            pltpu.sync_copy(cols.at[pl.ds(ib, _K)], idx_v)
            pltpu.sync_copy(rows.at[pl.ds(ib, _K)], row_v)
            pltpu.sync_copy(vals.at[pl.ds(ib, _K)], val_v)

            def fire_g(j, _):
                pltpu.async_copy(src.at[c].at[idx_v.at[j]], g_v.at[j], sem)
                return 0
            lax.fori_loop(0, _K, fire_g, 0)

            def drain(j, _):
                pltpu.make_async_copy(src.at[c].at[pl.ds(0, _SUB)],
                                      g_v.at[0], sem).wait()
                return 0
            lax.fori_loop(0, _K, drain, 0)

            def scale_j(j, _):
                def scale_m(mg, _):
                    m0 = mg * 16
                    vv = val_v[j, pl.ds(m0, 16)]
                    for t in range(16):
                        g_v[j, m0 + t, :] = g_v[j, m0 + t, :] * vv[t]
                    return 0
                lax.fori_loop(0, _SUB // 16, scale_m, 0)
                return 0
            lax.fori_loop(0, _K, scale_j, 0)

            def fire_s(j, _):
                pltpu.async_copy(g_v.at[j], acc.at[row_v.at[j]], sem, add=True)
                return 0
            lax.fori_loop(0, _K, fire_s, 0)
            lax.fori_loop(0, _K, drain, 0)
            return 0
        lax.fori_loop(0, _NCHUNK, chunk, 0)
        plsc.subcore_barrier()

        def cq(q, _):
            off = r0 + q * _QROWS
            pltpu.sync_copy(acc.at[pl.ds(off, _QROWS)], sa_v)
            if l < 2:
                pltpu.sync_copy(sa_v, xbuf.at[c].at[pl.ds(off, _QROWS)])
            prev = x0 if l == 0 else sum_o
            pltpu.sync_copy(prev.at[c].at[pl.ds(off, _QROWS)], sb_v)

            def addr(r, _):
                if l == 2:
                    sb_v[r, :] = (sb_v[r, :] + sa_v[r, :]) * 0.25
                else:
                    sb_v[r, :] = sb_v[r, :] + sa_v[r, :]
                return 0
            lax.fori_loop(0, _QROWS, addr, 0)
            pltpu.sync_copy(sb_v, sum_o.at[c].at[pl.ds(off, _QROWS)])
            return 0
        lax.fori_loop(0, _NQ, cq, 0)
        plsc.subcore_barrier()


_spmm3 = functools.partial(
    pl.kernel,
    mesh=plsc.VectorSubcoreMesh(core_axis_name="c", subcore_axis_name="s"),
    compiler_params=pltpu.CompilerParams(use_tc_tiling_on_sc=False),
    out_type=[
        jax.ShapeDtypeStruct((2, _NP, _H), jnp.float32),
        jax.ShapeDtypeStruct((2, _NP, _H), jnp.float32),
    ],
    scratch_types=[
        pltpu.VMEM((_K, _SUB, _H), jnp.float32),
        pltpu.VMEM((_K, _SUB), jnp.int32),
        pltpu.VMEM((_K, _SUB), jnp.int32),
        pltpu.VMEM((_K, _SUB), jnp.float32),
        pltpu.VMEM((_QROWS, _H), jnp.float32),
        pltpu.VMEM((_QROWS, _H), jnp.float32),
        pltpu.VMEM_SHARED((_NP, _H), jnp.float32),
        pltpu.SemaphoreType.DMA,
    ],
)(_spmm3_body)


def kernel(user_emb, item_emb, edge_values, edge_index):
    all_emb = jnp.concatenate(
        [user_emb, item_emb, jnp.zeros((_NP - _N, 32), jnp.float32)], axis=0)
    x0 = jnp.stack([all_emb[:, :_H], all_emb[:, _H:]], axis=0)
    rows = edge_index[0].astype(jnp.int32)
    cols = edge_index[1].astype(jnp.int32)
    pad = _EPAD - _E
    cols_p = jnp.concatenate([cols, jnp.zeros((pad,), jnp.int32)]).reshape(_EPAD // _SUB, _SUB)
    rows_p = jnp.concatenate([rows, jnp.zeros((pad,), jnp.int32)]).reshape(_EPAD // _SUB, _SUB)
    vals_p = jnp.concatenate([edge_values, jnp.zeros((pad,), jnp.float32)]).reshape(_EPAD // _SUB, _SUB)
    sum_o, _ = _spmm3(x0, cols_p, rows_p, vals_p)
    final = jnp.concatenate([sum_o[0, :_N], sum_o[1, :_N]], axis=1)
    return final[:_NUM_USERS], final[_NUM_USERS:]


# speedup vs baseline: 13.1124x; 1.1195x over previous
import functools

import jax
import jax.numpy as jnp
from jax import lax
from jax.experimental import pallas as pl
from jax.experimental.pallas import tpu as pltpu
from jax.experimental.pallas import tpu_sc as plsc

_NUM_USERS = 30000
_NUM_ITEMS = 70000
_N = _NUM_USERS + _NUM_ITEMS
_H = 16
_E = 1600000
_NS = 16
_SUB = 128
_CHUNK = 1024
_K = _CHUNK // _SUB
_NCHUNK = -(-_E // (_NS * _CHUNK))
_EPT = _NCHUNK * _CHUNK
_EPAD = _EPT * _NS
_IROWS_PT = _EPT // _SUB
_NP = 102400
_RPT = _NP // _NS
_QROWS = 400
_NQ = _RPT // _QROWS


def _spmm3_body(x0, edata, sum_o, xbuf, g_v, e_v, acc, sem):
    c = lax.axis_index("c")
    s = lax.axis_index("s")
    r0 = s * _RPT
    sa_v = g_v.at[pl.ds(0, _QROWS)]
    sb_v = g_v.at[pl.ds(512, _QROWS)]

    for l in range(3):
        def zrow(i, _):
            sa_v[i, :] = jnp.zeros((_H,), jnp.float32)
            return 0
        lax.fori_loop(0, _QROWS, zrow, 0)

        def zq(q, _):
            pltpu.sync_copy(sa_v, acc.at[pl.ds(r0 + q * _QROWS, _QROWS)])
            return 0
        lax.fori_loop(0, _NQ, zq, 0)
        plsc.subcore_barrier()

        src = x0 if l == 0 else xbuf

        def chunk(i, _):
            ib = s * _IROWS_PT + i * _K
            pltpu.sync_copy(edata.at[pl.ds(ib, _K)], e_v)

            def fire_g(j, _):
                pltpu.async_copy(src.at[c].at[e_v.at[j, 0]],
                                 g_v.at[pl.ds(j * _SUB, _SUB)], sem)
                return 0
            lax.fori_loop(0, _K, fire_g, 0)

            def drain(j, _):
                pltpu.make_async_copy(src.at[c].at[pl.ds(0, _SUB)],
                                      g_v.at[pl.ds(0, _SUB)], sem).wait()
                return 0
            lax.fori_loop(0, _K, drain, 0)

            def scale_j(j, _):
                def scale_m(mg, _):
                    m0 = mg * 16
                    vv = plsc.bitcast(e_v[j, 2, pl.ds(m0, 16)], jnp.float32)
                    e0 = j * _SUB + m0
                    for t in range(16):
                        g_v[e0 + t, :] = g_v[e0 + t, :] * vv[t]
                    return 0
                lax.fori_loop(0, _SUB // 16, scale_m, 0)
                return 0
            lax.fori_loop(0, _K, scale_j, 0)

            def fire_s(j, _):
                pltpu.async_copy(g_v.at[pl.ds(j * _SUB, _SUB)],
                                 acc.at[e_v.at[j, 1]], sem, add=True)
                return 0
            lax.fori_loop(0, _K, fire_s, 0)
            lax.fori_loop(0, _K, drain, 0)
            return 0
        lax.fori_loop(0, _NCHUNK, chunk, 0)
        plsc.subcore_barrier()

        def cq(q, _):
            off = r0 + q * _QROWS
            pltpu.sync_copy(acc.at[pl.ds(off, _QROWS)], sa_v)
            if l < 2:
                pltpu.sync_copy(sa_v, xbuf.at[c].at[pl.ds(off, _QROWS)])
            prev = x0 if l == 0 else sum_o
            pltpu.sync_copy(prev.at[c].at[pl.ds(off, _QROWS)], sb_v)

            def addr(r, _):
                if l == 2:
                    sb_v[r, :] = (sb_v[r, :] + sa_v[r, :]) * 0.25
                else:
                    sb_v[r, :] = sb_v[r, :] + sa_v[r, :]
                return 0
            lax.fori_loop(0, _QROWS, addr, 0)
            pltpu.sync_copy(sb_v, sum_o.at[c].at[pl.ds(off, _QROWS)])
            return 0
        lax.fori_loop(0, _NQ, cq, 0)
        plsc.subcore_barrier()


_spmm3 = functools.partial(
    pl.kernel,
    mesh=plsc.VectorSubcoreMesh(core_axis_name="c", subcore_axis_name="s"),
    compiler_params=pltpu.CompilerParams(use_tc_tiling_on_sc=False,
                                         needs_layout_passes=False),
    out_type=[
        jax.ShapeDtypeStruct((2, _NP, _H), jnp.float32),
        jax.ShapeDtypeStruct((2, _NP, _H), jnp.float32),
    ],
    scratch_types=[
        pltpu.VMEM((_CHUNK, _H), jnp.float32),
        pltpu.VMEM((_K, 3, _SUB), jnp.int32),
        pltpu.VMEM_SHARED((_NP, _H), jnp.float32),
        pltpu.SemaphoreType.DMA,
    ],
)(_spmm3_body)


def kernel(user_emb, item_emb, edge_values, edge_index):
    all_emb = jnp.concatenate(
        [user_emb, item_emb, jnp.zeros((_NP - _N, 32), jnp.float32)], axis=0)
    x0 = jnp.stack([all_emb[:, :_H], all_emb[:, _H:]], axis=0)
    rows = edge_index[0].astype(jnp.int32)
    cols = edge_index[1].astype(jnp.int32)
    pad = _EPAD - _E
    cols_p = jnp.concatenate([cols, jnp.zeros((pad,), jnp.int32)]).reshape(-1, _SUB)
    rows_p = jnp.concatenate([rows, jnp.zeros((pad,), jnp.int32)]).reshape(-1, _SUB)
    vals_p = jnp.concatenate([edge_values, jnp.zeros((pad,), jnp.float32)]).reshape(-1, _SUB)
    vals_i = jax.lax.bitcast_convert_type(vals_p, jnp.int32)
    edata = jnp.stack([cols_p, rows_p, vals_i], axis=1)
    sum_o, _ = _spmm3(x0, edata)
    final = jnp.concatenate([sum_o[0, :_N], sum_o[1, :_N]], axis=1)
    return final[:_NUM_USERS], final[_NUM_USERS:]


# speedup vs baseline: 15.2739x; 1.1648x over previous
import functools

import jax
import jax.numpy as jnp
from jax import lax
from jax.experimental import pallas as pl
from jax.experimental.pallas import tpu as pltpu
from jax.experimental.pallas import tpu_sc as plsc

_NUM_USERS = 30000
_NUM_ITEMS = 70000
_N = _NUM_USERS + _NUM_ITEMS
_H = 16
_E = 1600000
_NS = 16
_SUB = 128
_CHUNK = 512
_K = _CHUNK // _SUB
_NMACRO = 66
_NCHUNK = 3 * _NMACRO
_EPT = _NCHUNK * _CHUNK
_EPAD = _EPT * _NS
_IROWS_PT = _EPT // _SUB
_NP = 100352
_RPT = _NP // _NS
_QROWS = 448
_NQ = _RPT // _QROWS


def _spmm3_body(x0, edata, sum_o, xbuf,
                g0, g1, g2, e0, e1, e2, acc,
                sg0, sg1, sg2, ss0, ss1, ss2):
    c = lax.axis_index("c")
    s = lax.axis_index("s")
    r0 = s * _RPT
    gs = (g0, g1, g2)
    es = (e0, e1, e2)
    sgs = (sg0, sg1, sg2)
    sss = (ss0, ss1, ss2)
    sa_v = g0.at[pl.ds(0, _QROWS)]
    sb_v = g1.at[pl.ds(0, _QROWS)]

    def load_e(i, p):
        pltpu.sync_copy(edata.at[pl.ds(s * _IROWS_PT + i * _K, _K)], es[p])

    def fire_gather(src, p):
        for j in range(_K):
            pltpu.async_copy(src.at[c].at[es[p].at[j, 0]],
                             gs[p].at[pl.ds(j * _SUB, _SUB)], sgs[p])

    def drain(src, sem):
        for j in range(_K):
            pltpu.make_async_copy(src.at[c].at[pl.ds(0, _SUB)],
                                  gs[0].at[pl.ds(0, _SUB)], sem).wait()

    def scale(p):
        def scale_m(mg, _):
            m0 = mg * 16
            j = mg // (_SUB // 16)
            mm = (mg % (_SUB // 16)) * 16
            vv = plsc.bitcast(es[p][j, 2, pl.ds(mm, 16)], jnp.float32)
            e0_ = m0
            for t in range(16):
                gs[p][e0_ + t, :] = gs[p][e0_ + t, :] * vv[t]
            return 0
        lax.fori_loop(0, _CHUNK // 16, scale_m, 0)

    def fire_scatter(p):
        for j in range(_K):
            pltpu.async_copy(gs[p].at[pl.ds(j * _SUB, _SUB)],
                             acc.at[es[p].at[j, 1]], sss[p], add=True)

    for l in range(3):
        def zrow(i, _):
            sa_v[i, :] = jnp.zeros((_H,), jnp.float32)
            return 0
        lax.fori_loop(0, _QROWS, zrow, 0)

        def zq(q, _):
            pltpu.sync_copy(sa_v, acc.at[pl.ds(r0 + q * _QROWS, _QROWS)])
            return 0
        lax.fori_loop(0, _NQ, zq, 0)
        plsc.subcore_barrier()

        src = x0 if l == 0 else xbuf

        load_e(0, 0)
        fire_gather(src, 0)

        def macro(t, _):
            for q in range(3):
                i = 3 * t + q
                pn = (q + 1) % 3
                @pl.when(i >= 2)
                def _():
                    drain(src, sss[pn])
                @pl.when(i + 1 < _NCHUNK)
                def _():
                    load_e(i + 1, pn)
                    fire_gather(src, pn)
                drain(src, sgs[q])
                scale(q)
                fire_scatter(q)
            return 0
        lax.fori_loop(0, _NMACRO, macro, 0)
        drain(src, sss[(_NCHUNK - 2) % 3])
        drain(src, sss[(_NCHUNK - 1) % 3])
        plsc.subcore_barrier()

        def cq(q, _):
            off = r0 + q * _QROWS
            pltpu.sync_copy(acc.at[pl.ds(off, _QROWS)], sa_v)
            if l < 2:
                pltpu.sync_copy(sa_v, xbuf.at[c].at[pl.ds(off, _QROWS)])
            prev = x0 if l == 0 else sum_o
            pltpu.sync_copy(prev.at[c].at[pl.ds(off, _QROWS)], sb_v)

            def addr(r, _):
                if l == 2:
                    sb_v[r, :] = (sb_v[r, :] + sa_v[r, :]) * 0.25
                else:
                    sb_v[r, :] = sb_v[r, :] + sa_v[r, :]
                return 0
            lax.fori_loop(0, _QROWS, addr, 0)
            pltpu.sync_copy(sb_v, sum_o.at[c].at[pl.ds(off, _QROWS)])
            return 0
        lax.fori_loop(0, _NQ, cq, 0)
        plsc.subcore_barrier()


_spmm3 = functools.partial(
    pl.kernel,
    mesh=plsc.VectorSubcoreMesh(core_axis_name="c", subcore_axis_name="s"),
    compiler_params=pltpu.CompilerParams(use_tc_tiling_on_sc=False,
                                         needs_layout_passes=False),
    out_type=[
        jax.ShapeDtypeStruct((2, _NP, _H), jnp.float32),
        jax.ShapeDtypeStruct((2, _NP, _H), jnp.float32),
    ],
    scratch_types=[
        pltpu.VMEM((_CHUNK, _H), jnp.float32),
        pltpu.VMEM((_CHUNK, _H), jnp.float32),
        pltpu.VMEM((_CHUNK, _H), jnp.float32),
        pltpu.VMEM((_K, 3, _SUB), jnp.int32),
        pltpu.VMEM((_K, 3, _SUB), jnp.int32),
        pltpu.VMEM((_K, 3, _SUB), jnp.int32),
        pltpu.VMEM_SHARED((_NP, _H), jnp.float32),
        pltpu.SemaphoreType.DMA,
        pltpu.SemaphoreType.DMA,
        pltpu.SemaphoreType.DMA,
        pltpu.SemaphoreType.DMA,
        pltpu.SemaphoreType.DMA,
        pltpu.SemaphoreType.DMA,
    ],
)(_spmm3_body)


def kernel(user_emb, item_emb, edge_values, edge_index):
    all_emb = jnp.concatenate(
        [user_emb, item_emb, jnp.zeros((_NP - _N, 32), jnp.float32)], axis=0)
    x0 = jnp.stack([all_emb[:, :_H], all_emb[:, _H:]], axis=0)
    rows = edge_index[0].astype(jnp.int32)
    cols = edge_index[1].astype(jnp.int32)
    pad = _EPAD - _E
    cols_p = jnp.concatenate([cols, jnp.zeros((pad,), jnp.int32)]).reshape(-1, _SUB)
    rows_p = jnp.concatenate([rows, jnp.zeros((pad,), jnp.int32)]).reshape(-1, _SUB)
    vals_p = jnp.concatenate([edge_values, jnp.zeros((pad,), jnp.float32)]).reshape(-1, _SUB)
    vals_i = jax.lax.bitcast_convert_type(vals_p, jnp.int32)
    edata = jnp.stack([cols_p, rows_p, vals_i], axis=1)
    sum_o, _ = _spmm3(x0, edata)
    final = jnp.concatenate([sum_o[0, :_N], sum_o[1, :_N]], axis=1)
    return final[:_NUM_USERS], final[_NUM_USERS:]


# speedup vs baseline: 15.5704x; 1.0194x over previous
import functools

import jax
import jax.numpy as jnp
from jax import lax
from jax.experimental import pallas as pl
from jax.experimental.pallas import tpu as pltpu
from jax.experimental.pallas import tpu_sc as plsc

_NUM_USERS = 30000
_NUM_ITEMS = 70000
_N = _NUM_USERS + _NUM_ITEMS
_H = 16
_E = 1600000
_NS = 16
_SUB = 128
_CHUNK = 512
_K = _CHUNK // _SUB
_NMACRO = 66
_NCHUNK = 3 * _NMACRO
_EPT = _NCHUNK * _CHUNK
_EPAD = _EPT * _NS
_IROWS_PT = _EPT // _SUB
_NP = 100352
_RPT = _NP // _NS
_QROWS = 448
_NQ = _RPT // _QROWS


def _spmm3_body(x0, edata, sum_o, xbuf,
                g0, g1, g2, e0, e1, e2, acc,
                sg0, sg1, sg2, ss0, ss1, ss2):
    c = lax.axis_index("c")
    s = lax.axis_index("s")
    r0 = s * _RPT
    gs = (g0, g1, g2)
    es = (e0, e1, e2)
    sgs = (sg0, sg1, sg2)
    sss = (ss0, ss1, ss2)
    sa_v = g0.at[pl.ds(0, _QROWS)]
    sb_v = g1.at[pl.ds(0, _QROWS)]

    def load_e(i, p):
        pltpu.sync_copy(edata.at[pl.ds(s * _IROWS_PT + i * _K, _K)], es[p])

    def fire_gather(src, p):
        for j in range(_K):
            pltpu.async_copy(src.at[c].at[es[p].at[j, 0]],
                             gs[p].at[pl.ds(j * _SUB, _SUB)], sgs[p])

    def drain(src, sem):
        for j in range(_K):
            pltpu.make_async_copy(src.at[c].at[pl.ds(0, _SUB)],
                                  gs[0].at[pl.ds(0, _SUB)], sem).wait()

    def scale(p):
        @plsc.parallel_loop(0, _CHUNK // 16, unroll=2)
        def _(mg):
            m0 = mg * 16
            j = mg // (_SUB // 16)
            mm = (mg % (_SUB // 16)) * 16
            vv = plsc.bitcast(es[p][j, 2, pl.ds(mm, 16)], jnp.float32)
            for t in range(16):
                gs[p][m0 + t, :] = gs[p][m0 + t, :] * vv[t]

    def fire_scatter(p):
        for j in range(_K):
            pltpu.async_copy(gs[p].at[pl.ds(j * _SUB, _SUB)],
                             acc.at[es[p].at[j, 1]], sss[p], add=True)

    for l in range(3):
        def zrow(i, _):
            sa_v[i, :] = jnp.zeros((_H,), jnp.float32)
            return 0
        lax.fori_loop(0, _QROWS, zrow, 0)

        def zq(q, _):
            pltpu.sync_copy(sa_v, acc.at[pl.ds(r0 + q * _QROWS, _QROWS)])
            return 0
        lax.fori_loop(0, _NQ, zq, 0)
        plsc.subcore_barrier()

        src = x0 if l == 0 else xbuf

        load_e(0, 0)
        fire_gather(src, 0)

        def macro(t, _):
            for q in range(3):
                i = 3 * t + q
                pn = (q + 1) % 3
                @pl.when(i >= 2)
                def _():
                    drain(src, sss[pn])
                @pl.when(i + 1 < _NCHUNK)
                def _():
                    load_e(i + 1, pn)
                    fire_gather(src, pn)
                drain(src, sgs[q])
                scale(q)
                fire_scatter(q)
            return 0
        lax.fori_loop(0, _NMACRO, macro, 0)
        drain(src, sss[(_NCHUNK - 2) % 3])
        drain(src, sss[(_NCHUNK - 1) % 3])
        plsc.subcore_barrier()

        def cq(q, _):
            off = r0 + q * _QROWS
            pltpu.sync_copy(acc.at[pl.ds(off, _QROWS)], sa_v)
            if l < 2:
                pltpu.sync_copy(sa_v, xbuf.at[c].at[pl.ds(off, _QROWS)])
            prev = x0 if l == 0 else sum_o
            pltpu.sync_copy(prev.at[c].at[pl.ds(off, _QROWS)], sb_v)

            def addr(r, _):
                if l == 2:
                    sb_v[r, :] = (sb_v[r, :] + sa_v[r, :]) * 0.25
                else:
                    sb_v[r, :] = sb_v[r, :] + sa_v[r, :]
                return 0
            lax.fori_loop(0, _QROWS, addr, 0)
            pltpu.sync_copy(sb_v, sum_o.at[c].at[pl.ds(off, _QROWS)])
            return 0
        lax.fori_loop(0, _NQ, cq, 0)
        plsc.subcore_barrier()


_spmm3 = functools.partial(
    pl.kernel,
    mesh=plsc.VectorSubcoreMesh(core_axis_name="c", subcore_axis_name="s"),
    compiler_params=pltpu.CompilerParams(use_tc_tiling_on_sc=False,
                                         needs_layout_passes=False),
    out_type=[
        jax.ShapeDtypeStruct((2, _NP, _H), jnp.float32),
        jax.ShapeDtypeStruct((2, _NP, _H), jnp.float32),
    ],
    scratch_types=[
        pltpu.VMEM((_CHUNK, _H), jnp.float32),
        pltpu.VMEM((_CHUNK, _H), jnp.float32),
        pltpu.VMEM((_CHUNK, _H), jnp.float32),
        pltpu.VMEM((_K, 3, _SUB), jnp.int32),
        pltpu.VMEM((_K, 3, _SUB), jnp.int32),
        pltpu.VMEM((_K, 3, _SUB), jnp.int32),
        pltpu.VMEM_SHARED((_NP, _H), jnp.float32),
        pltpu.SemaphoreType.DMA,
        pltpu.SemaphoreType.DMA,
        pltpu.SemaphoreType.DMA,
        pltpu.SemaphoreType.DMA,
        pltpu.SemaphoreType.DMA,
        pltpu.SemaphoreType.DMA,
    ],
)(_spmm3_body)


def kernel(user_emb, item_emb, edge_values, edge_index):
    all_emb = jnp.concatenate(
        [user_emb, item_emb, jnp.zeros((_NP - _N, 32), jnp.float32)], axis=0)
    x0 = jnp.stack([all_emb[:, :_H], all_emb[:, _H:]], axis=0)
    rows = edge_index[0].astype(jnp.int32)
    cols = edge_index[1].astype(jnp.int32)
    pad = _EPAD - _E
    cols_p = jnp.concatenate([cols, jnp.zeros((pad,), jnp.int32)]).reshape(-1, _SUB)
    rows_p = jnp.concatenate([rows, jnp.zeros((pad,), jnp.int32)]).reshape(-1, _SUB)
    vals_p = jnp.concatenate([edge_values, jnp.zeros((pad,), jnp.float32)]).reshape(-1, _SUB)
    vals_i = jax.lax.bitcast_convert_type(vals_p, jnp.int32)
    edata = jnp.stack([cols_p, rows_p, vals_i], axis=1)
    sum_o, _ = _spmm3(x0, edata)
    final = jnp.concatenate([sum_o[0, :_N], sum_o[1, :_N]], axis=1)
    return final[:_NUM_USERS], final[_NUM_USERS:]
